# Initial kernel scaffold; baseline (speedup 1.0000x reference)
import jax
import jax.numpy as jnp
from jax.experimental import pallas as pl


def kernel(w, beta, x, y, particle_id):
    raise NotImplementedError("write your pallas kernel here")


# speedup vs baseline: 7.5201x; 7.5201x over previous
import jax
import jax.numpy as jnp
from jax.experimental import pallas as pl
from jax.experimental.pallas import tpu as pltpu

_Q_MIN = 0.01
_N = 100000
_LANES = 128
_ROWS = 784
_NPAD = _ROWS * _LANES


def _loss_kernel(beta_ref, pid_ref, x0_ref, x1_ref, x2_ref, out_ref):
    beta = beta_ref[...]
    pid = pid_ref[...]
    x0 = x0_ref[...]
    x1 = x1_ref[...]
    x2 = x2_ref[...]

    at = 0.5 * jnp.log((1.0 + beta) / (1.0 - beta))
    q = at * at + _Q_MIN
    ridx = jax.lax.broadcasted_iota(jnp.int32, (_ROWS, _LANES), 0)
    cidx = jax.lax.broadcasted_iota(jnp.int32, (_ROWS, _LANES), 1)
    flat = ridx * _LANES + cidx
    valid = flat < _N
    q = jnp.where(valid, q, 0.0)
    flat_f = flat.astype(jnp.float32)

    def body(p, acc):
        mask = pid == p
        masked_q = jnp.where(mask, q, -1.0)
        qm = jnp.max(masked_q)
        qa = jnp.where(qm > 0.0, qm, 0.0)
        match = jnp.logical_and(mask, q == qm)
        mi = jnp.min(jnp.where(match, flat_f, 3.0e38))
        sel = flat_f == mi
        a0 = jnp.sum(jnp.where(sel, x0, 0.0))
        a1 = jnp.sum(jnp.where(sel, x1, 0.0))
        a2 = jnp.sum(jnp.where(sel, x2, 0.0))
        d0 = x0 - a0
        d1 = x1 - a1
        d2c = x2 - a2
        dist2 = d0 * d0 + d1 * d1 + d2c * d2c
        norm = jnp.sqrt(dist2)
        rep = jnp.maximum(1.0 - norm, 0.0)
        val = jnp.where(mask, dist2, 10.0 * rep)
        return acc + qa * jnp.sum(q * val)

    total = jax.lax.fori_loop(1, 50, body, jnp.float32(0.0))
    out_ref[0, 0] = total * (1.0 / _N)


def kernel(w, beta, x, y, particle_id):
    del w, y
    pid = particle_id.reshape(-1).astype(jnp.int32)
    pad = _NPAD - _N
    beta_p = jnp.pad(beta, (0, pad)).reshape(_ROWS, _LANES)
    pid_p = jnp.pad(pid, (0, pad)).reshape(_ROWS, _LANES)
    x_p = jnp.pad(x.astype(jnp.float32), ((0, pad), (0, 0)))
    x0 = x_p[:, 0].reshape(_ROWS, _LANES)
    x1 = x_p[:, 1].reshape(_ROWS, _LANES)
    x2 = x_p[:, 2].reshape(_ROWS, _LANES)

    out = pl.pallas_call(
        _loss_kernel,
        out_shape=jax.ShapeDtypeStruct((1, 1), jnp.float32),
        in_specs=[pl.BlockSpec((_ROWS, _LANES), lambda: (0, 0))] * 5,
        out_specs=pl.BlockSpec(memory_space=pltpu.SMEM),
    )(beta_p, pid_p, x0, x1, x2)
    return out[0, 0]
